# baseline (speedup 1.0000x reference)
import functools
import math

import jax
import jax.numpy as jnp
import numpy as np
from jax import lax
from jax.experimental import pallas as pl
from jax.experimental.pallas import tpu as pltpu
from jax.experimental.pallas import tpu_sc as plsc

N = 10000
E = 160000
NB = 10
MAX_RADIUS = 2.0
D_IN = 16
D_EDGE = 4
D_SH = 9
D_EA = D_EDGE + D_SH
D_HID = 144
FC_HID = 100

NC = 2
NS = 16
NW = NC * NS
K = 128
E_PAD = 163840
NBLK = E_PAD // (NW * K)
N_PAD = 10240
ROWS_PER_SID = N_PAD // NS
ZCH = 64

_SQ3 = float(np.sqrt(3.0))
_SQ15 = float(np.sqrt(15.0))
_SQ5H = float(np.sqrt(5.0) / 2.0)
_SQ15H = float(np.sqrt(15.0) / 2.0)
_VALUES = np.linspace(0.0, MAX_RADIUS, NB + 2)[1:-1].astype(np.float32)
_STEP = float(_VALUES[1] - _VALUES[0])
_SQNB = float(np.sqrt(float(NB)))


def _mesh():
    return plsc.VectorSubcoreMesh(core_axis_name="c", subcore_axis_name="s")


_SC_PARAMS = pltpu.CompilerParams(needs_layout_passes=False,
                                  use_tc_tiling_on_sc=False)


def _make_edge_vec_kernel():
    @functools.partial(
        pl.kernel,
        mesh=_mesh(),
        out_type=jax.ShapeDtypeStruct((E_PAD * 3,), jnp.float32),
        compiler_params=_SC_PARAMS,
        scratch_types=[
            pltpu.VMEM((N_PAD * 3,), jnp.float32),
            pltpu.VMEM((K,), jnp.int32),
            pltpu.VMEM((K,), jnp.int32),
            pltpu.VMEM((K * 3,), jnp.float32),
        ],
    )
    def edge_vec_kernel(pos_hbm, src_hbm, dst_hbm, ev_hbm, pos_v, src_v, dst_v, out_v):
        cid = lax.axis_index("c")
        sid = lax.axis_index("s")
        wid = cid * NS + sid
        pltpu.sync_copy(pos_hbm, pos_v)

        def block_body(b, _):
            base = (wid * NBLK + b) * K
            pltpu.sync_copy(src_hbm.at[pl.ds(base, K)], src_v)
            pltpu.sync_copy(dst_hbm.at[pl.ds(base, K)], dst_v)
            lane = lax.iota(jnp.int32, 16)
            for k in range(K // 16):
                s3 = src_v[pl.ds(k * 16, 16)] * 3
                d3 = dst_v[pl.ds(k * 16, 16)] * 3
                row3 = (lane + k * 16) * 3
                for comp in range(3):
                    ps = plsc.load_gather(pos_v, [s3 + comp])
                    pd = plsc.load_gather(pos_v, [d3 + comp])
                    plsc.store_scatter(out_v, [row3 + comp], ps - pd)
            pltpu.sync_copy(out_v, ev_hbm.at[pl.ds(base * 3, K * 3)])
            return ()

        lax.fori_loop(0, NBLK, block_body, (), unroll=False)

    return edge_vec_kernel


D_LO = 80
D_HI = D_HID - D_LO


def _make_layer_kernel(d_c, with_degw):
    out_type = [jax.ShapeDtypeStruct((NC, N_PAD, d_c), jnp.float32)]
    if with_degw:
        out_type.append(jax.ShapeDtypeStruct((NC, N_PAD), jnp.float32))
    scratch = [
        pltpu.VMEM((K,), jnp.int32),
        pltpu.VMEM((K,), jnp.int32),
        pltpu.VMEM((K, d_c), jnp.float32),
        pltpu.VMEM((K, d_c), jnp.float32),
        pltpu.VMEM((K, d_c), jnp.float32),
        pltpu.VMEM((ZCH, d_c), jnp.float32),
        pltpu.VMEM_SHARED((N_PAD, d_c), jnp.float32),
        pltpu.SemaphoreType.DMA,
    ]
    if with_degw:
        scratch.append(pltpu.VMEM((K,), jnp.float32))
        scratch.append(pltpu.VMEM_SHARED((N_PAD,), jnp.float32))

    def body(*refs):
        if with_degw:
            (src_hbm, dst_hbm, hw_hbm, ew_hbm, c_hbm, z144_hbm, z1_hbm, ew2_hbm,
             agg_out, dw_out,
             src_v, dst_v, g_v, ew_v, c_v, z_v, agg_s, sem, ew2_v, dw_s) = refs
        else:
            (src_hbm, dst_hbm, hw_hbm, ew_hbm, c_hbm, z144_hbm,
             agg_out,
             src_v, dst_v, g_v, ew_v, c_v, z_v, agg_s, sem) = refs
        cid = lax.axis_index("c")
        sid = lax.axis_index("s")
        wid = cid * NS + sid

        pltpu.sync_copy(z144_hbm, z_v)
        for j in range(ROWS_PER_SID // ZCH):
            pltpu.sync_copy(z_v, agg_s.at[pl.ds(sid * ROWS_PER_SID + j * ZCH, ZCH), :])
        if with_degw:
            @pl.when(sid == 0)
            def _():
                pltpu.sync_copy(z1_hbm, dw_s)
        plsc.subcore_barrier()

        def block_body(b, _):
            base = (wid * NBLK + b) * K
            pltpu.sync_copy(src_hbm.at[pl.ds(base, K)], src_v)
            pltpu.sync_copy(dst_hbm.at[pl.ds(base, K)], dst_v)
            gather = pltpu.async_copy(hw_hbm.at[src_v], g_v, sem)
            pltpu.sync_copy(ew_hbm.at[pl.ds(base, K), :], ew_v)
            pltpu.sync_copy(c_hbm.at[pl.ds(base, K), :], c_v)
            if with_degw:
                pltpu.sync_copy(ew2_hbm.at[pl.ds(base, K)], ew2_v)
                pltpu.sync_copy(ew2_v, dw_s.at[src_v], add=True)
            gather.wait()

            def edge_body(i, _):
                for j in range(d_c // 16):
                    sl = pl.ds(j * 16, 16)
                    g_v[i, sl] = g_v[i, sl] * ew_v[i, sl] + c_v[i, sl]
                return ()

            lax.fori_loop(0, K, edge_body, (), unroll=False)
            pltpu.sync_copy(g_v, agg_s.at[dst_v], add=True)
            return ()

        lax.fori_loop(0, NBLK, block_body, (), unroll=False)
        plsc.subcore_barrier()
        rows = pl.ds(sid * ROWS_PER_SID, ROWS_PER_SID)
        pltpu.sync_copy(agg_s.at[rows, :], agg_out.at[cid, rows, :])
        if with_degw:
            pltpu.sync_copy(dw_s.at[rows], dw_out.at[cid, rows])

    return functools.partial(
        pl.kernel, mesh=_mesh(), out_type=out_type, scratch_types=scratch,
        compiler_params=_SC_PARAMS,
    )(body)


BE = 512


def _geom(ev_ref, eat_ref, pid):
    v = ev_ref[...]
    length = jnp.sqrt(jnp.sum(v * v, axis=1, keepdims=True))
    unit = v / (length + 1e-9)
    x = unit[:, 0:1]
    y = unit[:, 1:2]
    z = unit[:, 2:3]
    sh = jnp.concatenate([
        jnp.ones_like(x), _SQ3 * unit,
        _SQ15 * x * y, _SQ15 * y * z, _SQ5H * (3.0 * z * z - 1.0),
        _SQ15 * x * z, _SQ15H * (x * x - y * y),
    ], axis=1)
    ea = jnp.concatenate([eat_ref[...], sh], axis=1)
    vals = (lax.broadcasted_iota(jnp.int32, (1, NB), 1).astype(jnp.float32)
            + 1.0) * _STEP
    diff = (length - vals) / _STEP
    emb = jnp.cos((np.pi / 2.0) * diff) * ((diff < 1.0) & (diff > -1.0)) * _SQNB
    ids = pid * BE + lax.broadcasted_iota(jnp.int32, (BE, 1), 0)
    mask = (ids < E).astype(jnp.float32)
    return ea, emb, mask


def _coeffs(ea, emb, mask, Wf1, bf, Wf2, We):
    s = jax.nn.silu(jnp.dot(emb, Wf1[...], preferred_element_type=jnp.float32)
                    + bf[...][None, :])
    ew = jnp.dot(s, Wf2[...], preferred_element_type=jnp.float32)
    c = jnp.dot(ea, We[...], preferred_element_type=jnp.float32) * ew
    return ew * mask, c * mask


def _tc_coeff01_body(ev_ref, eat_ref, Wf1, bf, Wf2, We,
                     ewa_ref, ewb_ref, ca_ref, cb_ref):
    ea, emb, mask = _geom(ev_ref, eat_ref, pl.program_id(0))
    ew, c = _coeffs(ea, emb, mask, Wf1, bf, Wf2, We)
    ewa_ref[...], ewb_ref[...] = ew[:, :D_LO], ew[:, D_LO:]
    ca_ref[...], cb_ref[...] = c[:, :D_LO], c[:, D_LO:]


def _tc_coeff01(ev, eat, Wf1, bf, Wf2, We):
    full = lambda shape: pl.BlockSpec(shape, lambda i: tuple(0 for _ in shape))
    row = lambda d: pl.BlockSpec((BE, d), lambda i: (i, 0))
    return pl.pallas_call(
        _tc_coeff01_body,
        grid=(E_PAD // BE,),
        in_specs=[row(3), row(D_EDGE),
                  full((NB, FC_HID)), full((FC_HID,)), full((FC_HID, D_HID)),
                  full((D_EA, D_HID))],
        out_specs=[row(D_LO), row(D_HI), row(D_LO), row(D_HI)],
        out_shape=[jax.ShapeDtypeStruct((E_PAD, D_LO), jnp.float32),
                   jax.ShapeDtypeStruct((E_PAD, D_HI), jnp.float32)] * 2,
    )(ev, eat, Wf1, bf, Wf2, We)


def _tc_coeff2_body(ev_ref, eat_ref, Wf1, bf, Wf2, We, ew2_ref, c2_ref):
    ea, emb, mask = _geom(ev_ref, eat_ref, pl.program_id(0))
    ew, c = _coeffs(ea, emb, mask, Wf1, bf, Wf2, We)
    t = jnp.transpose(jnp.concatenate([ew, c], axis=1))
    ew2_ref[...] = t[0:1, :][None]
    c2_ref[...] = t[1:2, :][None]


def _tc_coeff2(ev, eat, Wf1, bf, Wf2, We):
    full = lambda shape: pl.BlockSpec(shape, lambda i: tuple(0 for _ in shape))
    return pl.pallas_call(
        _tc_coeff2_body,
        grid=(E_PAD // BE,),
        in_specs=[pl.BlockSpec((BE, 3), lambda i: (i, 0)),
                  pl.BlockSpec((BE, D_EDGE), lambda i: (i, 0)),
                  full((NB, FC_HID)), full((FC_HID,)), full((FC_HID, 1)),
                  full((D_EA, 1))],
        out_specs=[pl.BlockSpec((1, 1, BE), lambda i: (i, 0, 0))] * 2,
        out_shape=[jax.ShapeDtypeStruct((E_PAD // BE, 1, BE), jnp.float32)] * 2,
    )(ev, eat, Wf1, bf, Wf2, We)


NROW = 1000


def _split_outs(hW, hWs, o1a_ref, o1b_ref, o2_ref):
    o1a_ref[...], o1b_ref[...] = hW[:, :D_LO], hW[:, D_LO:]
    o2_ref[...] = hWs


def _node_out_specs(dout, split):
    if split:
        return ([pl.BlockSpec((NROW, D_LO), lambda i: (i, 0)),
                 pl.BlockSpec((NROW, D_HI), lambda i: (i, 0)),
                 pl.BlockSpec((NROW, dout), lambda i: (i, 0))],
                [jax.ShapeDtypeStruct((N, D_LO), jnp.float32),
                 jax.ShapeDtypeStruct((N, D_HI), jnp.float32),
                 jax.ShapeDtypeStruct((N, dout), jnp.float32)])
    return ([pl.BlockSpec((NROW, dout), lambda i: (i, 0))] * 2,
            [jax.ShapeDtypeStruct((N, dout), jnp.float32)] * 2)


def _tc_pre(h, Wn, Ws):
    din, dout = Wn.shape
    full = lambda shape: pl.BlockSpec(shape, lambda i: tuple(0 for _ in shape))
    out_specs, out_shape = _node_out_specs(dout, True)

    def body(h_ref, Wn_ref, Ws_ref, o1a_ref, o1b_ref, o2_ref):
        h = h_ref[...]
        hW = jnp.dot(h, Wn_ref[...], preferred_element_type=jnp.float32)
        hWs = jnp.dot(h, Ws_ref[...], preferred_element_type=jnp.float32)
        _split_outs(hW, hWs, o1a_ref, o1b_ref, o2_ref)

    return pl.pallas_call(
        body,
        grid=(N // NROW,),
        in_specs=[pl.BlockSpec((NROW, din), lambda i: (i, 0)),
                  full((din, dout)), full((din, dout))],
        out_specs=out_specs, out_shape=out_shape,
    )(h, Wn, Ws)


def _tc_post(pa, pb, hWs, attr, Wn, Ws, split):
    din, dout = Wn.shape
    full = lambda shape: pl.BlockSpec(shape, lambda i: tuple(0 for _ in shape))
    out_specs, out_shape = _node_out_specs(dout, split)

    def body(pa_ref, pb_ref, hWs_ref, attr_ref, Wn_ref, Ws_ref, *outs):
        agg = jnp.concatenate([pa_ref[0] + pa_ref[1],
                               pb_ref[0] + pb_ref[1]], axis=1) * 0.25
        h = jax.nn.silu(hWs_ref[...] * attr_ref[...] + agg)
        hW = jnp.dot(h, Wn_ref[...], preferred_element_type=jnp.float32)
        hWs2 = jnp.dot(h, Ws_ref[...], preferred_element_type=jnp.float32)
        if split:
            _split_outs(hW, hWs2, *outs)
        else:
            outs[0][...] = hW
            outs[1][...] = hWs2

    return pl.pallas_call(
        body,
        grid=(N // NROW,),
        in_specs=[pl.BlockSpec((NC, NROW, D_LO), lambda i: (0, i, 0)),
                  pl.BlockSpec((NC, NROW, D_HI), lambda i: (0, i, 0)),
                  pl.BlockSpec((NROW, din), lambda i: (i, 0)),
                  pl.BlockSpec((NROW, 1), lambda i: (i, 0)),
                  full((din, dout)), full((din, dout))],
        out_specs=out_specs, out_shape=out_shape,
    )(pa, pb, hWs, attr, Wn, Ws)


def _tc_final_body(hW2_ref, hWs2_ref, attr_ref, dw_ref, c2_ref, out_ref):
    term1 = jnp.sum(hWs2_ref[...] * attr_ref[...])
    degw = dw_ref[0, :N, :] + dw_ref[1, :N, :]
    term2 = jnp.sum(hW2_ref[...] * degw)
    term3 = jnp.sum(c2_ref[...])
    total = (term1 + (term2 + term3) * 0.25) / 100.0
    out_ref[...] = jnp.reshape(total, (1, 1))


def _tc_final(hW2, hWs2, attr, dw, c2):
    full = lambda shape: pl.BlockSpec(shape, lambda: tuple(0 for _ in shape))
    return pl.pallas_call(
        _tc_final_body,
        in_specs=[full((N, 1)), full((N, 1)), full((N, 1)),
                  full((NC, N_PAD, 1)), full((E_PAD // BE, 1, BE))],
        out_specs=full((1, 1)),
        out_shape=jax.ShapeDtypeStruct((1, 1), jnp.float32),
    )(hW2, hWs2, attr, dw, c2)


def kernel(pos, node_input, node_attr, edge_attr, edge_index, batch,
           Wn0, We0, Wf1_0, bf0, Wf2_0, Ws0,
           Wn1, We1, Wf1_1, bf1, Wf2_1, Ws1,
           Wn2, We2, Wf1_2, bf2, Wf2_2, Ws2):
    pad = E_PAD - E
    src = jnp.concatenate([edge_index[0], jnp.zeros((pad,), jnp.int32)])
    dst = jnp.concatenate([edge_index[1], jnp.full((pad,), N, jnp.int32)])
    eat = jnp.concatenate([edge_attr, jnp.zeros((pad, D_EDGE), jnp.float32)])
    pos_pad = jnp.concatenate([pos, jnp.zeros((N_PAD - N, 3), jnp.float32)])
    z_lo = jnp.zeros((ZCH, D_LO), jnp.float32)
    z_hi = jnp.zeros((ZCH, D_HI), jnp.float32)
    z1 = jnp.zeros((N_PAD,), jnp.float32)

    ev = _make_edge_vec_kernel()(jnp.reshape(pos_pad, (N_PAD * 3,)), src, dst)
    ev = jnp.reshape(ev, (E_PAD, 3))
    ew0a, ew0b, c0a, c0b = _tc_coeff01(ev, eat, Wf1_0, bf0, Wf2_0, We0)
    ew2, c2 = _tc_coeff2(ev, eat, Wf1_2, bf2, Wf2_2, We2)
    ew1a, ew1b, c1a, c1b = _tc_coeff01(ev, eat, Wf1_1, bf1, Wf2_1, We1)
    hW0a, hW0b, hWs0 = _tc_pre(node_input, Wn0, Ws0)

    lo_dw = _make_layer_kernel(D_LO, with_degw=True)
    hi = _make_layer_kernel(D_HI, with_degw=False)
    lo = _make_layer_kernel(D_LO, with_degw=False)

    p0a, dw = lo_dw(src, dst, hW0a, ew0a, c0a, z_lo, z1,
                    jnp.reshape(ew2, (E_PAD,)))
    (p0b,) = hi(src, dst, hW0b, ew0b, c0b, z_hi)
    hW1a, hW1b, hWs1 = _tc_post(p0a, p0b, hWs0, node_attr, Wn1, Ws1, True)

    (p1a,) = lo(src, dst, hW1a, ew1a, c1a, z_lo)
    (p1b,) = hi(src, dst, hW1b, ew1b, c1b, z_hi)
    hW2, hWs2 = _tc_post(p1a, p1b, hWs1, node_attr, Wn2, Ws2, False)

    return _tc_final(hW2, hWs2, node_attr,
                     jnp.reshape(dw, (NC, N_PAD, 1)), c2)


# speedup vs baseline: 1.4728x; 1.4728x over previous
import functools
import math

import jax
import jax.numpy as jnp
import numpy as np
from jax import lax
from jax.experimental import pallas as pl
from jax.experimental.pallas import tpu as pltpu
from jax.experimental.pallas import tpu_sc as plsc

N = 10000
E = 160000
NB = 10
MAX_RADIUS = 2.0
D_IN = 16
D_EDGE = 4
D_SH = 9
D_EA = D_EDGE + D_SH
D_HID = 144
FC_HID = 100

NC = 2
NS = 16
NW = NC * NS
K = 128
E_PAD = 163840
NBLK = E_PAD // (NW * K)
N_PAD = 10240
ROWS_PER_SID = N_PAD // NS
ZCH = 64

_SQ3 = float(np.sqrt(3.0))
_SQ15 = float(np.sqrt(15.0))
_SQ5H = float(np.sqrt(5.0) / 2.0)
_SQ15H = float(np.sqrt(15.0) / 2.0)
_VALUES = np.linspace(0.0, MAX_RADIUS, NB + 2)[1:-1].astype(np.float32)
_STEP = float(_VALUES[1] - _VALUES[0])
_SQNB = float(np.sqrt(float(NB)))


def _mesh():
    return plsc.VectorSubcoreMesh(core_axis_name="c", subcore_axis_name="s")


_SC_PARAMS = pltpu.CompilerParams(needs_layout_passes=False,
                                  use_tc_tiling_on_sc=False)


def _make_edge_vec_kernel():
    @functools.partial(
        pl.kernel,
        mesh=_mesh(),
        out_type=jax.ShapeDtypeStruct((E_PAD * 3,), jnp.float32),
        compiler_params=_SC_PARAMS,
        scratch_types=[
            pltpu.VMEM((N_PAD * 3,), jnp.float32),
            pltpu.VMEM((K,), jnp.int32),
            pltpu.VMEM((K,), jnp.int32),
            pltpu.VMEM((K * 3,), jnp.float32),
        ],
    )
    def edge_vec_kernel(pos_hbm, src_hbm, dst_hbm, ev_hbm, pos_v, src_v, dst_v, out_v):
        cid = lax.axis_index("c")
        sid = lax.axis_index("s")
        wid = cid * NS + sid
        pltpu.sync_copy(pos_hbm, pos_v)

        def block_body(b, _):
            base = (wid * NBLK + b) * K
            pltpu.sync_copy(src_hbm.at[pl.ds(base, K)], src_v)
            pltpu.sync_copy(dst_hbm.at[pl.ds(base, K)], dst_v)
            lane = lax.iota(jnp.int32, 16)
            for k in range(K // 16):
                s3 = src_v[pl.ds(k * 16, 16)] * 3
                d3 = dst_v[pl.ds(k * 16, 16)] * 3
                row3 = (lane + k * 16) * 3
                for comp in range(3):
                    ps = plsc.load_gather(pos_v, [s3 + comp])
                    pd = plsc.load_gather(pos_v, [d3 + comp])
                    plsc.store_scatter(out_v, [row3 + comp], ps - pd)
            pltpu.sync_copy(out_v, ev_hbm.at[pl.ds(base * 3, K * 3)])
            return ()

        lax.fori_loop(0, NBLK, block_body, (), unroll=False)

    return edge_vec_kernel


D_LO = 80
D_HI = D_HID - D_LO


def _make_layer_kernel(d_c, with_degw):
    out_type = [jax.ShapeDtypeStruct((NC, N_PAD, d_c), jnp.float32)]
    if with_degw:
        out_type.append(jax.ShapeDtypeStruct((NC, N_PAD), jnp.float32))
    buf = [pltpu.VMEM((K,), jnp.int32),
           pltpu.VMEM((K,), jnp.int32),
           pltpu.VMEM((K, d_c), jnp.float32),
           pltpu.VMEM((K, d_c), jnp.float32),
           pltpu.VMEM((K, d_c), jnp.float32)]
    scratch = buf + buf + [
        pltpu.VMEM((ZCH, d_c), jnp.float32),
        pltpu.VMEM_SHARED((N_PAD, d_c), jnp.float32),
        pltpu.SemaphoreType.DMA,
        pltpu.SemaphoreType.DMA,
    ]
    if with_degw:
        scratch.append(pltpu.VMEM((K,), jnp.float32))
        scratch.append(pltpu.VMEM((K,), jnp.float32))
        scratch.append(pltpu.VMEM_SHARED((N_PAD,), jnp.float32))

    def body(*refs):
        if with_degw:
            (src_hbm, dst_hbm, hw_hbm, ew_hbm, c_hbm, z144_hbm, z1_hbm, ew2_hbm,
             agg_out, dw_out,
             src0, dst0, g0, ew0, c0, src1, dst1, g1, ew1, c1,
             z_v, agg_s, sem0, sem1, ew2_0, ew2_1, dw_s) = refs
        else:
            (src_hbm, dst_hbm, hw_hbm, ew_hbm, c_hbm, z144_hbm,
             agg_out,
             src0, dst0, g0, ew0, c0, src1, dst1, g1, ew1, c1,
             z_v, agg_s, sem0, sem1) = refs
            ew2_0 = ew2_1 = None
        cid = lax.axis_index("c")
        sid = lax.axis_index("s")
        wid = cid * NS + sid

        pltpu.sync_copy(z144_hbm, z_v)
        for j in range(ROWS_PER_SID // ZCH):
            pltpu.sync_copy(z_v, agg_s.at[pl.ds(sid * ROWS_PER_SID + j * ZCH, ZCH), :])
        if with_degw:
            @pl.when(sid == 0)
            def _():
                pltpu.sync_copy(z1_hbm, dw_s)
        plsc.subcore_barrier()

        def issue(b, src_v, dst_v, g_v, ew_v, c_v, ew2_v, sem):
            base = (wid * NBLK + b) * K
            pltpu.sync_copy(src_hbm.at[pl.ds(base, K)], src_v)
            pltpu.sync_copy(dst_hbm.at[pl.ds(base, K)], dst_v)
            if with_degw:
                pltpu.sync_copy(ew2_hbm.at[pl.ds(base, K)], ew2_v)
            pltpu.async_copy(hw_hbm.at[src_v], g_v, sem)
            pltpu.async_copy(ew_hbm.at[pl.ds(base, K), :], ew_v, sem)
            pltpu.async_copy(c_hbm.at[pl.ds(base, K), :], c_v, sem)

        def consume(b, src_v, dst_v, g_v, ew_v, c_v, ew2_v, sem):
            base = (wid * NBLK + b) * K
            pltpu.make_async_copy(hw_hbm.at[src_v], g_v, sem).wait()
            pltpu.make_async_copy(ew_hbm.at[pl.ds(base, K), :], ew_v, sem).wait()
            pltpu.make_async_copy(c_hbm.at[pl.ds(base, K), :], c_v, sem).wait()
            if with_degw:
                pltpu.sync_copy(ew2_v, dw_s.at[src_v], add=True)

            def edge_body(i, _):
                for j in range(d_c // 16):
                    sl = pl.ds(j * 16, 16)
                    g_v[i, sl] = g_v[i, sl] * ew_v[i, sl] + c_v[i, sl]
                return ()

            lax.fori_loop(0, K, edge_body, (), unroll=False)
            pltpu.sync_copy(g_v, agg_s.at[dst_v], add=True)

        buf0 = (src0, dst0, g0, ew0, c0, ew2_0, sem0)
        buf1 = (src1, dst1, g1, ew1, c1, ew2_1, sem1)
        issue(0, *buf0)

        def pair_body(t, _):
            b0 = 2 * t
            issue(b0 + 1, *buf1)
            consume(b0, *buf0)

            @pl.when(b0 + 2 < NBLK)
            def _():
                issue(b0 + 2, *buf0)

            consume(b0 + 1, *buf1)
            return ()

        lax.fori_loop(0, NBLK // 2, pair_body, (), unroll=False)
        plsc.subcore_barrier()
        rows = pl.ds(sid * ROWS_PER_SID, ROWS_PER_SID)
        pltpu.sync_copy(agg_s.at[rows, :], agg_out.at[cid, rows, :])
        if with_degw:
            pltpu.sync_copy(dw_s.at[rows], dw_out.at[cid, rows])

    return functools.partial(
        pl.kernel, mesh=_mesh(), out_type=out_type, scratch_types=scratch,
        compiler_params=_SC_PARAMS,
    )(body)


BE = 512


def _geom(ev_ref, eat_ref, pid):
    v = ev_ref[...]
    length = jnp.sqrt(jnp.sum(v * v, axis=1, keepdims=True))
    unit = v / (length + 1e-9)
    x = unit[:, 0:1]
    y = unit[:, 1:2]
    z = unit[:, 2:3]
    sh = jnp.concatenate([
        jnp.ones_like(x), _SQ3 * unit,
        _SQ15 * x * y, _SQ15 * y * z, _SQ5H * (3.0 * z * z - 1.0),
        _SQ15 * x * z, _SQ15H * (x * x - y * y),
    ], axis=1)
    ea = jnp.concatenate([eat_ref[...], sh], axis=1)
    vals = (lax.broadcasted_iota(jnp.int32, (1, NB), 1).astype(jnp.float32)
            + 1.0) * _STEP
    diff = (length - vals) / _STEP
    theta = length * (np.pi / (2.0 * _STEP))
    ct, st = jnp.cos(theta), jnp.sin(theta)
    jm = lax.broadcasted_iota(jnp.int32, (1, NB), 1) % 4
    wave = jnp.where(jm == 0, st, jnp.where(jm == 1, -ct,
                     jnp.where(jm == 2, -st, ct)))
    emb = wave * ((diff < 1.0) & (diff > -1.0)) * _SQNB
    ids = pid * BE + lax.broadcasted_iota(jnp.int32, (BE, 1), 0)
    mask = (ids < E).astype(jnp.float32)
    return ea, emb, mask


def _bf(a):
    return a.astype(jnp.bfloat16)


def _coeffs(ea, emb, mask, Wf1, bf, Wf2, We):
    z = jnp.dot(_bf(emb), _bf(Wf1[...]), preferred_element_type=jnp.float32)
    s = jax.nn.silu(z + bf[...][None, :])
    ew = jnp.dot(_bf(s), _bf(Wf2[...]), preferred_element_type=jnp.float32)
    c = jnp.dot(_bf(ea), _bf(We[...]), preferred_element_type=jnp.float32) * ew
    return ew * mask, c * mask


def _tc_coeff_body(ev_ref, eat_ref,
                   Wf1_0, bf0, Wf2_0, We0, Wf1_1, bf1, Wf2_1, We1,
                   Wf1_2, bf2, Wf2_2, We2,
                   ew0a_ref, ew0b_ref, c0a_ref, c0b_ref,
                   ew1a_ref, ew1b_ref, c1a_ref, c1b_ref,
                   ew2_ref, c2_ref):
    ea, emb, mask = _geom(ev_ref, eat_ref, pl.program_id(0))
    ew0, c0 = _coeffs(ea, emb, mask, Wf1_0, bf0, Wf2_0, We0)
    ew0a_ref[...], ew0b_ref[...] = ew0[:, :D_LO], ew0[:, D_LO:]
    c0a_ref[...], c0b_ref[...] = c0[:, :D_LO], c0[:, D_LO:]
    ew1, c1 = _coeffs(ea, emb, mask, Wf1_1, bf1, Wf2_1, We1)
    ew1a_ref[...], ew1b_ref[...] = ew1[:, :D_LO], ew1[:, D_LO:]
    c1a_ref[...], c1b_ref[...] = c1[:, :D_LO], c1[:, D_LO:]
    ew2, c2 = _coeffs(ea, emb, mask, Wf1_2, bf2, Wf2_2, We2)
    t = jnp.transpose(jnp.concatenate([ew2, c2], axis=1))
    ew2_ref[...] = t[0:1, :][None]
    c2_ref[...] = t[1:2, :][None]


def _tc_coeffs(ev, eat, Wf1_0, bf0, Wf2_0, We0, Wf1_1, bf1, Wf2_1, We1,
               Wf1_2, bf2, Wf2_2, We2):
    full = lambda shape: pl.BlockSpec(shape, lambda i: tuple(0 for _ in shape))
    row = lambda d: pl.BlockSpec((BE, d), lambda i: (i, 0))
    return pl.pallas_call(
        _tc_coeff_body,
        grid=(E_PAD // BE,),
        in_specs=[row(3), row(D_EDGE),
                  full((NB, FC_HID)), full((FC_HID,)), full((FC_HID, D_HID)),
                  full((D_EA, D_HID)),
                  full((NB, FC_HID)), full((FC_HID,)), full((FC_HID, D_HID)),
                  full((D_EA, D_HID)),
                  full((NB, FC_HID)), full((FC_HID,)), full((FC_HID, 1)),
                  full((D_EA, 1))],
        out_specs=[row(D_LO), row(D_HI), row(D_LO), row(D_HI),
                   row(D_LO), row(D_HI), row(D_LO), row(D_HI),
                   pl.BlockSpec((1, 1, BE), lambda i: (i, 0, 0)),
                   pl.BlockSpec((1, 1, BE), lambda i: (i, 0, 0))],
        out_shape=[jax.ShapeDtypeStruct((E_PAD, D_LO), jnp.float32),
                   jax.ShapeDtypeStruct((E_PAD, D_HI), jnp.float32)] * 4
        + [jax.ShapeDtypeStruct((E_PAD // BE, 1, BE), jnp.float32)] * 2,
    )(ev, eat, Wf1_0, bf0, Wf2_0, We0, Wf1_1, bf1, Wf2_1, We1,
      Wf1_2, bf2, Wf2_2, We2)


NROW = 1000


def _split_outs(hW, hWs, o1a_ref, o1b_ref, o2_ref):
    o1a_ref[...], o1b_ref[...] = hW[:, :D_LO], hW[:, D_LO:]
    o2_ref[...] = hWs


def _node_out_specs(dout, split):
    if split:
        return ([pl.BlockSpec((NROW, D_LO), lambda i: (i, 0)),
                 pl.BlockSpec((NROW, D_HI), lambda i: (i, 0)),
                 pl.BlockSpec((NROW, dout), lambda i: (i, 0))],
                [jax.ShapeDtypeStruct((N, D_LO), jnp.float32),
                 jax.ShapeDtypeStruct((N, D_HI), jnp.float32),
                 jax.ShapeDtypeStruct((N, dout), jnp.float32)])
    return ([pl.BlockSpec((NROW, dout), lambda i: (i, 0))] * 2,
            [jax.ShapeDtypeStruct((N, dout), jnp.float32)] * 2)


def _tc_pre(h, Wn, Ws):
    din, dout = Wn.shape
    full = lambda shape: pl.BlockSpec(shape, lambda i: tuple(0 for _ in shape))
    out_specs, out_shape = _node_out_specs(dout, True)

    def body(h_ref, Wn_ref, Ws_ref, o1a_ref, o1b_ref, o2_ref):
        h = h_ref[...]
        hW = jnp.dot(h, Wn_ref[...], preferred_element_type=jnp.float32)
        hWs = jnp.dot(h, Ws_ref[...], preferred_element_type=jnp.float32)
        _split_outs(hW, hWs, o1a_ref, o1b_ref, o2_ref)

    return pl.pallas_call(
        body,
        grid=(N // NROW,),
        in_specs=[pl.BlockSpec((NROW, din), lambda i: (i, 0)),
                  full((din, dout)), full((din, dout))],
        out_specs=out_specs, out_shape=out_shape,
    )(h, Wn, Ws)


def _tc_post(pa, pb, hWs, attr, Wn, Ws, split):
    din, dout = Wn.shape
    full = lambda shape: pl.BlockSpec(shape, lambda i: tuple(0 for _ in shape))
    out_specs, out_shape = _node_out_specs(dout, split)

    def body(pa_ref, pb_ref, hWs_ref, attr_ref, Wn_ref, Ws_ref, *outs):
        agg = jnp.concatenate([pa_ref[0] + pa_ref[1],
                               pb_ref[0] + pb_ref[1]], axis=1) * 0.25
        h = jax.nn.silu(hWs_ref[...] * attr_ref[...] + agg)
        hW = jnp.dot(h, Wn_ref[...], preferred_element_type=jnp.float32)
        hWs2 = jnp.dot(h, Ws_ref[...], preferred_element_type=jnp.float32)
        if split:
            _split_outs(hW, hWs2, *outs)
        else:
            outs[0][...] = hW
            outs[1][...] = hWs2

    return pl.pallas_call(
        body,
        grid=(N // NROW,),
        in_specs=[pl.BlockSpec((NC, NROW, D_LO), lambda i: (0, i, 0)),
                  pl.BlockSpec((NC, NROW, D_HI), lambda i: (0, i, 0)),
                  pl.BlockSpec((NROW, din), lambda i: (i, 0)),
                  pl.BlockSpec((NROW, 1), lambda i: (i, 0)),
                  full((din, dout)), full((din, dout))],
        out_specs=out_specs, out_shape=out_shape,
    )(pa, pb, hWs, attr, Wn, Ws)


def _tc_final_body(hW2_ref, hWs2_ref, attr_ref, dw_ref, c2_ref, out_ref):
    term1 = jnp.sum(hWs2_ref[...] * attr_ref[...])
    degw = dw_ref[0, :N, :] + dw_ref[1, :N, :]
    term2 = jnp.sum(hW2_ref[...] * degw)
    term3 = jnp.sum(c2_ref[...])
    total = (term1 + (term2 + term3) * 0.25) / 100.0
    out_ref[...] = jnp.reshape(total, (1, 1))


def _tc_final(hW2, hWs2, attr, dw, c2):
    full = lambda shape: pl.BlockSpec(shape, lambda: tuple(0 for _ in shape))
    return pl.pallas_call(
        _tc_final_body,
        in_specs=[full((N, 1)), full((N, 1)), full((N, 1)),
                  full((NC, N_PAD, 1)), full((E_PAD // BE, 1, BE))],
        out_specs=full((1, 1)),
        out_shape=jax.ShapeDtypeStruct((1, 1), jnp.float32),
    )(hW2, hWs2, attr, dw, c2)


def kernel(pos, node_input, node_attr, edge_attr, edge_index, batch,
           Wn0, We0, Wf1_0, bf0, Wf2_0, Ws0,
           Wn1, We1, Wf1_1, bf1, Wf2_1, Ws1,
           Wn2, We2, Wf1_2, bf2, Wf2_2, Ws2):
    pad = E_PAD - E
    src = jnp.concatenate([edge_index[0], jnp.zeros((pad,), jnp.int32)])
    dst = jnp.concatenate([edge_index[1], jnp.full((pad,), N, jnp.int32)])
    eat = jnp.concatenate([edge_attr, jnp.zeros((pad, D_EDGE), jnp.float32)])
    pos_pad = jnp.concatenate([pos, jnp.zeros((N_PAD - N, 3), jnp.float32)])
    z_lo = jnp.zeros((ZCH, D_LO), jnp.float32)
    z_hi = jnp.zeros((ZCH, D_HI), jnp.float32)
    z1 = jnp.zeros((N_PAD,), jnp.float32)

    ev = _make_edge_vec_kernel()(jnp.reshape(pos_pad, (N_PAD * 3,)), src, dst)
    ev = jnp.reshape(ev, (E_PAD, 3))
    (ew0a, ew0b, c0a, c0b, ew1a, ew1b, c1a, c1b, ew2, c2) = _tc_coeffs(
        ev, eat, Wf1_0, bf0, Wf2_0, We0, Wf1_1, bf1, Wf2_1, We1,
        Wf1_2, bf2, Wf2_2, We2)
    hW0a, hW0b, hWs0 = _tc_pre(node_input, Wn0, Ws0)

    lo_dw = _make_layer_kernel(D_LO, with_degw=True)
    hi = _make_layer_kernel(D_HI, with_degw=False)
    lo = _make_layer_kernel(D_LO, with_degw=False)

    p0a, dw = lo_dw(src, dst, hW0a, ew0a, c0a, z_lo, z1,
                    jnp.reshape(ew2, (E_PAD,)))
    (p0b,) = hi(src, dst, hW0b, ew0b, c0b, z_hi)
    hW1a, hW1b, hWs1 = _tc_post(p0a, p0b, hWs0, node_attr, Wn1, Ws1, True)

    (p1a,) = lo(src, dst, hW1a, ew1a, c1a, z_lo)
    (p1b,) = hi(src, dst, hW1b, ew1b, c1b, z_hi)
    hW2, hWs2 = _tc_post(p1a, p1b, hWs1, node_attr, Wn2, Ws2, False)

    return _tc_final(hW2, hWs2, node_attr,
                     jnp.reshape(dw, (NC, N_PAD, 1)), c2)


# speedup vs baseline: 1.9131x; 1.2990x over previous
import functools
import math

import jax
import jax.numpy as jnp
import numpy as np
from jax import lax
from jax.experimental import pallas as pl
from jax.experimental.pallas import tpu as pltpu
from jax.experimental.pallas import tpu_sc as plsc

N = 10000
E = 160000
NB = 10
MAX_RADIUS = 2.0
D_IN = 16
D_EDGE = 4
D_SH = 9
D_EA = D_EDGE + D_SH
D_HID = 144
FC_HID = 100

NC = 2
NS = 16
NW = NC * NS
K = 128
E_PAD = 163840
NBLK = E_PAD // (NW * K)
N_PAD = 10240
ROWS_PER_SID = N_PAD // NS
ZCH = 64

_SQ3 = float(np.sqrt(3.0))
_SQ15 = float(np.sqrt(15.0))
_SQ5H = float(np.sqrt(5.0) / 2.0)
_SQ15H = float(np.sqrt(15.0) / 2.0)
_VALUES = np.linspace(0.0, MAX_RADIUS, NB + 2)[1:-1].astype(np.float32)
_STEP = float(_VALUES[1] - _VALUES[0])
_SQNB = float(np.sqrt(float(NB)))


def _mesh():
    return plsc.VectorSubcoreMesh(core_axis_name="c", subcore_axis_name="s")


_SC_PARAMS = pltpu.CompilerParams(needs_layout_passes=False,
                                  use_tc_tiling_on_sc=False)


def _make_edge_vec_kernel():
    @functools.partial(
        pl.kernel,
        mesh=_mesh(),
        out_type=jax.ShapeDtypeStruct((3, E_PAD), jnp.float32),
        compiler_params=_SC_PARAMS,
        scratch_types=[
            pltpu.VMEM((N_PAD * 3,), jnp.float32),
            pltpu.VMEM((K,), jnp.int32),
            pltpu.VMEM((K,), jnp.int32),
            pltpu.VMEM((3, K), jnp.float32),
        ],
    )
    def edge_vec_kernel(pos_hbm, src_hbm, dst_hbm, ev_hbm, pos_v, src_v, dst_v, out_v):
        cid = lax.axis_index("c")
        sid = lax.axis_index("s")
        wid = cid * NS + sid
        pltpu.sync_copy(pos_hbm, pos_v)

        def block_body(b, _):
            base = (wid * NBLK + b) * K
            pltpu.sync_copy(src_hbm.at[pl.ds(base, K)], src_v)
            pltpu.sync_copy(dst_hbm.at[pl.ds(base, K)], dst_v)
            for k in range(K // 16):
                s3 = src_v[pl.ds(k * 16, 16)] * 3
                d3 = dst_v[pl.ds(k * 16, 16)] * 3
                for comp in range(3):
                    ps = plsc.load_gather(pos_v, [s3 + comp])
                    pd = plsc.load_gather(pos_v, [d3 + comp])
                    out_v[comp, pl.ds(k * 16, 16)] = ps - pd
            pltpu.sync_copy(out_v, ev_hbm.at[:, pl.ds(base, K)])
            return ()

        lax.fori_loop(0, NBLK, block_body, (), unroll=False)

    return edge_vec_kernel


D_LO = 80
D_HI = D_HID - D_LO


def _make_layer_kernel(d_c, with_degw):
    out_type = [jax.ShapeDtypeStruct((NC, N_PAD, d_c), jnp.float32)]
    if with_degw:
        out_type.append(jax.ShapeDtypeStruct((NC, N_PAD), jnp.float32))
    buf = [pltpu.VMEM((K,), jnp.int32),
           pltpu.VMEM((K,), jnp.int32),
           pltpu.VMEM((K, d_c), jnp.float32),
           pltpu.VMEM((K, d_c), jnp.float32),
           pltpu.VMEM((K, d_c), jnp.float32)]
    scratch = buf + buf + [
        pltpu.VMEM((ZCH, d_c), jnp.float32),
        pltpu.VMEM_SHARED((N_PAD, d_c), jnp.float32),
        pltpu.SemaphoreType.DMA,
        pltpu.SemaphoreType.DMA,
    ]
    if with_degw:
        scratch.append(pltpu.VMEM((K,), jnp.float32))
        scratch.append(pltpu.VMEM((K,), jnp.float32))
        scratch.append(pltpu.VMEM_SHARED((N_PAD,), jnp.float32))

    def body(*refs):
        if with_degw:
            (src_hbm, dst_hbm, hw_hbm, ew_hbm, c_hbm, z144_hbm, z1_hbm, ew2_hbm,
             agg_out, dw_out,
             src0, dst0, g0, ew0, c0, src1, dst1, g1, ew1, c1,
             z_v, agg_s, sem0, sem1, ew2_0, ew2_1, dw_s) = refs
        else:
            (src_hbm, dst_hbm, hw_hbm, ew_hbm, c_hbm, z144_hbm,
             agg_out,
             src0, dst0, g0, ew0, c0, src1, dst1, g1, ew1, c1,
             z_v, agg_s, sem0, sem1) = refs
            ew2_0 = ew2_1 = None
        cid = lax.axis_index("c")
        sid = lax.axis_index("s")
        wid = cid * NS + sid

        pltpu.sync_copy(z144_hbm, z_v)
        for j in range(ROWS_PER_SID // ZCH):
            pltpu.sync_copy(z_v, agg_s.at[pl.ds(sid * ROWS_PER_SID + j * ZCH, ZCH), :])
        if with_degw:
            @pl.when(sid == 0)
            def _():
                pltpu.sync_copy(z1_hbm, dw_s)
        plsc.subcore_barrier()

        def issue(b, src_v, dst_v, g_v, ew_v, c_v, ew2_v, sem):
            base = (wid * NBLK + b) * K
            pltpu.sync_copy(src_hbm.at[pl.ds(base, K)], src_v)
            pltpu.sync_copy(dst_hbm.at[pl.ds(base, K)], dst_v)
            if with_degw:
                pltpu.sync_copy(ew2_hbm.at[pl.ds(base, K)], ew2_v)
            pltpu.async_copy(hw_hbm.at[src_v], g_v, sem)
            pltpu.async_copy(ew_hbm.at[pl.ds(base, K), :], ew_v, sem)
            pltpu.async_copy(c_hbm.at[pl.ds(base, K), :], c_v, sem)

        def consume(b, src_v, dst_v, g_v, ew_v, c_v, ew2_v, sem):
            base = (wid * NBLK + b) * K
            pltpu.make_async_copy(hw_hbm.at[src_v], g_v, sem).wait()
            pltpu.make_async_copy(ew_hbm.at[pl.ds(base, K), :], ew_v, sem).wait()
            pltpu.make_async_copy(c_hbm.at[pl.ds(base, K), :], c_v, sem).wait()
            if with_degw:
                pltpu.sync_copy(ew2_v, dw_s.at[src_v], add=True)

            def edge_body(i, _):
                for j in range(d_c // 16):
                    sl = pl.ds(j * 16, 16)
                    g_v[i, sl] = g_v[i, sl] * ew_v[i, sl] + c_v[i, sl]
                return ()

            lax.fori_loop(0, K, edge_body, (), unroll=False)
            pltpu.sync_copy(g_v, agg_s.at[dst_v], add=True)

        buf0 = (src0, dst0, g0, ew0, c0, ew2_0, sem0)
        buf1 = (src1, dst1, g1, ew1, c1, ew2_1, sem1)
        issue(0, *buf0)

        def pair_body(t, _):
            b0 = 2 * t
            issue(b0 + 1, *buf1)
            consume(b0, *buf0)

            @pl.when(b0 + 2 < NBLK)
            def _():
                issue(b0 + 2, *buf0)

            consume(b0 + 1, *buf1)
            return ()

        lax.fori_loop(0, NBLK // 2, pair_body, (), unroll=False)
        plsc.subcore_barrier()
        rows = pl.ds(sid * ROWS_PER_SID, ROWS_PER_SID)
        pltpu.sync_copy(agg_s.at[rows, :], agg_out.at[cid, rows, :])
        if with_degw:
            pltpu.sync_copy(dw_s.at[rows], dw_out.at[cid, rows])

    return functools.partial(
        pl.kernel, mesh=_mesh(), out_type=out_type, scratch_types=scratch,
        compiler_params=_SC_PARAMS,
    )(body)


BE = 512


def _bf(a):
    return a.astype(jnp.bfloat16)


def _geom_t(ev_ref, eat_ref, pid):
    v = ev_ref[...]
    x0, y0, z0 = v[0:1, :], v[1:2, :], v[2:3, :]
    length = jnp.sqrt(x0 * x0 + y0 * y0 + z0 * z0)
    inv = 1.0 / (length + 1e-9)
    x, y, z = x0 * inv, y0 * inv, z0 * inv
    sh = jnp.concatenate([
        jnp.ones_like(x), _SQ3 * x, _SQ3 * y, _SQ3 * z,
        _SQ15 * x * y, _SQ15 * y * z, _SQ5H * (3.0 * z * z - 1.0),
        _SQ15 * x * z, _SQ15H * (x * x - y * y),
    ], axis=0)
    ea = jnp.concatenate([eat_ref[...], sh], axis=0)
    vals = (lax.broadcasted_iota(jnp.int32, (NB, 1), 0).astype(jnp.float32)
            + 1.0) * _STEP
    diff = (length - vals) / _STEP
    emb = jnp.cos((np.pi / 2.0) * diff) * ((diff < 1.0) & (diff > -1.0)) * _SQNB
    ids = pid * BE + lax.broadcasted_iota(jnp.int32, (1, BE), 1)
    mask = (ids < E).astype(jnp.float32)
    return ea, emb, mask


def _coeffs(ea, emb, mask, Wf1, bf, Wf2, We):
    z = jnp.dot(_bf(emb), _bf(Wf1[...]), preferred_element_type=jnp.float32)
    s = jax.nn.silu(z + bf[...][None, :])
    ew = jnp.dot(_bf(s), _bf(Wf2[...]), preferred_element_type=jnp.float32)
    c = jnp.dot(_bf(ea), _bf(We[...]), preferred_element_type=jnp.float32) * ew
    return ew * mask, c * mask


def _tc_coeff_body(ev_ref, eat_ref,
                   Wf1_0, bf0, Wf2_0, We0, Wf1_1, bf1, Wf2_1, We1,
                   Wf1_2, bf2, Wf2_2, We2,
                   ew0a_ref, ew0b_ref, c0a_ref, c0b_ref,
                   ew1a_ref, ew1b_ref, c1a_ref, c1b_ref,
                   ew2_ref, c2_ref):
    pid = pl.program_id(0)
    eaT, embT, _ = _geom_t(ev_ref, eat_ref, pid)
    ea = jnp.transpose(eaT)
    emb = jnp.transpose(embT)
    ids = pid * BE + lax.broadcasted_iota(jnp.int32, (BE, 1), 0)
    mask = (ids < E).astype(jnp.float32)
    ew0, c0 = _coeffs(ea, emb, mask, Wf1_0, bf0, Wf2_0, We0)
    ew0a_ref[...], ew0b_ref[...] = ew0[:, :D_LO], ew0[:, D_LO:]
    c0a_ref[...], c0b_ref[...] = c0[:, :D_LO], c0[:, D_LO:]
    ew1, c1 = _coeffs(ea, emb, mask, Wf1_1, bf1, Wf2_1, We1)
    ew1a_ref[...], ew1b_ref[...] = ew1[:, :D_LO], ew1[:, D_LO:]
    c1a_ref[...], c1b_ref[...] = c1[:, :D_LO], c1[:, D_LO:]
    ew2, c2 = _coeffs(ea, emb, mask, Wf1_2, bf2, Wf2_2, We2)
    t = jnp.transpose(jnp.concatenate([ew2, c2], axis=1))
    ew2_ref[...] = t[0:1, :][None]
    c2_ref[...] = t[1:2, :][None]


def _tc_coeffs(ev, eat, Wf1_0, bf0, Wf2_0, We0, Wf1_1, bf1, Wf2_1, We1,
               Wf1_2, bf2, Wf2_2, We2):
    full = lambda shape: pl.BlockSpec(shape, lambda i: tuple(0 for _ in shape))
    row = lambda d: pl.BlockSpec((BE, d), lambda i: (i, 0))
    col = lambda d: pl.BlockSpec((d, BE), lambda i: (0, i))
    return pl.pallas_call(
        _tc_coeff_body,
        grid=(E_PAD // BE,),
        in_specs=[col(3), col(D_EDGE),
                  full((NB, FC_HID)), full((FC_HID,)), full((FC_HID, D_HID)),
                  full((D_EA, D_HID)),
                  full((NB, FC_HID)), full((FC_HID,)), full((FC_HID, D_HID)),
                  full((D_EA, D_HID)),
                  full((NB, FC_HID)), full((FC_HID,)), full((FC_HID, 1)),
                  full((D_EA, 1))],
        out_specs=[row(D_LO), row(D_HI), row(D_LO), row(D_HI),
                   row(D_LO), row(D_HI), row(D_LO), row(D_HI),
                   pl.BlockSpec((1, 1, BE), lambda i: (i, 0, 0)),
                   pl.BlockSpec((1, 1, BE), lambda i: (i, 0, 0))],
        out_shape=[jax.ShapeDtypeStruct((E_PAD, D_LO), jnp.float32),
                   jax.ShapeDtypeStruct((E_PAD, D_HI), jnp.float32)] * 4
        + [jax.ShapeDtypeStruct((E_PAD // BE, 1, BE), jnp.float32)] * 2,
    )(ev, eat, Wf1_0, bf0, Wf2_0, We0, Wf1_1, bf1, Wf2_1, We1,
      Wf1_2, bf2, Wf2_2, We2)


NROW = 1000


def _split_outs(hW, hWs, o1a_ref, o1b_ref, o2_ref):
    o1a_ref[...], o1b_ref[...] = hW[:, :D_LO], hW[:, D_LO:]
    o2_ref[...] = hWs


def _node_out_specs(dout, split):
    if split:
        return ([pl.BlockSpec((NROW, D_LO), lambda i: (i, 0)),
                 pl.BlockSpec((NROW, D_HI), lambda i: (i, 0)),
                 pl.BlockSpec((NROW, dout), lambda i: (i, 0))],
                [jax.ShapeDtypeStruct((N, D_LO), jnp.float32),
                 jax.ShapeDtypeStruct((N, D_HI), jnp.float32),
                 jax.ShapeDtypeStruct((N, dout), jnp.float32)])
    return ([pl.BlockSpec((NROW, dout), lambda i: (i, 0))] * 2,
            [jax.ShapeDtypeStruct((N, dout), jnp.float32)] * 2)


def _tc_pre(h, Wn, Ws):
    din, dout = Wn.shape
    full = lambda shape: pl.BlockSpec(shape, lambda i: tuple(0 for _ in shape))
    out_specs, out_shape = _node_out_specs(dout, True)

    def body(h_ref, Wn_ref, Ws_ref, o1a_ref, o1b_ref, o2_ref):
        h = h_ref[...]
        hW = jnp.dot(h, Wn_ref[...], preferred_element_type=jnp.float32)
        hWs = jnp.dot(h, Ws_ref[...], preferred_element_type=jnp.float32)
        _split_outs(hW, hWs, o1a_ref, o1b_ref, o2_ref)

    return pl.pallas_call(
        body,
        grid=(N // NROW,),
        in_specs=[pl.BlockSpec((NROW, din), lambda i: (i, 0)),
                  full((din, dout)), full((din, dout))],
        out_specs=out_specs, out_shape=out_shape,
    )(h, Wn, Ws)


def _tc_post(pa, pb, hWs, attr, Wn, Ws, split):
    din, dout = Wn.shape
    full = lambda shape: pl.BlockSpec(shape, lambda i: tuple(0 for _ in shape))
    out_specs, out_shape = _node_out_specs(dout, split)

    def body(pa_ref, pb_ref, hWs_ref, attr_ref, Wn_ref, Ws_ref, *outs):
        agg = jnp.concatenate([pa_ref[0] + pa_ref[1],
                               pb_ref[0] + pb_ref[1]], axis=1) * 0.25
        h = jax.nn.silu(hWs_ref[...] * attr_ref[...] + agg)
        hW = jnp.dot(h, Wn_ref[...], preferred_element_type=jnp.float32)
        hWs2 = jnp.dot(h, Ws_ref[...], preferred_element_type=jnp.float32)
        if split:
            _split_outs(hW, hWs2, *outs)
        else:
            outs[0][...] = hW
            outs[1][...] = hWs2

    return pl.pallas_call(
        body,
        grid=(N // NROW,),
        in_specs=[pl.BlockSpec((NC, NROW, D_LO), lambda i: (0, i, 0)),
                  pl.BlockSpec((NC, NROW, D_HI), lambda i: (0, i, 0)),
                  pl.BlockSpec((NROW, din), lambda i: (i, 0)),
                  pl.BlockSpec((NROW, 1), lambda i: (i, 0)),
                  full((din, dout)), full((din, dout))],
        out_specs=out_specs, out_shape=out_shape,
    )(pa, pb, hWs, attr, Wn, Ws)


def _tc_final_body(hW2_ref, hWs2_ref, attr_ref, dw_ref, c2_ref, out_ref):
    term1 = jnp.sum(hWs2_ref[...] * attr_ref[...])
    degw = dw_ref[0, :N, :] + dw_ref[1, :N, :]
    term2 = jnp.sum(hW2_ref[...] * degw)
    term3 = jnp.sum(c2_ref[...])
    total = (term1 + (term2 + term3) * 0.25) / 100.0
    out_ref[...] = jnp.reshape(total, (1, 1))


def _tc_final(hW2, hWs2, attr, dw, c2):
    full = lambda shape: pl.BlockSpec(shape, lambda: tuple(0 for _ in shape))
    return pl.pallas_call(
        _tc_final_body,
        in_specs=[full((N, 1)), full((N, 1)), full((N, 1)),
                  full((NC, N_PAD, 1)), full((E_PAD // BE, 1, BE))],
        out_specs=full((1, 1)),
        out_shape=jax.ShapeDtypeStruct((1, 1), jnp.float32),
    )(hW2, hWs2, attr, dw, c2)


def kernel(pos, node_input, node_attr, edge_attr, edge_index, batch,
           Wn0, We0, Wf1_0, bf0, Wf2_0, Ws0,
           Wn1, We1, Wf1_1, bf1, Wf2_1, Ws1,
           Wn2, We2, Wf1_2, bf2, Wf2_2, Ws2):
    pad = E_PAD - E
    src = jnp.concatenate([edge_index[0], jnp.zeros((pad,), jnp.int32)])
    dst = jnp.concatenate([edge_index[1], jnp.full((pad,), N, jnp.int32)])
    eat = jnp.concatenate([jnp.transpose(edge_attr),
                           jnp.zeros((D_EDGE, pad), jnp.float32)], axis=1)
    pos_pad = jnp.concatenate([pos, jnp.zeros((N_PAD - N, 3), jnp.float32)])
    z_lo = jnp.zeros((ZCH, D_LO), jnp.float32)
    z_hi = jnp.zeros((ZCH, D_HI), jnp.float32)
    z1 = jnp.zeros((N_PAD,), jnp.float32)

    ev = _make_edge_vec_kernel()(jnp.reshape(pos_pad, (N_PAD * 3,)), src, dst)
    (ew0a, ew0b, c0a, c0b, ew1a, ew1b, c1a, c1b, ew2, c2) = _tc_coeffs(
        ev, eat, Wf1_0, bf0, Wf2_0, We0, Wf1_1, bf1, Wf2_1, We1,
        Wf1_2, bf2, Wf2_2, We2)
    hW0a, hW0b, hWs0 = _tc_pre(node_input, Wn0, Ws0)

    lo_dw = _make_layer_kernel(D_LO, with_degw=True)
    hi = _make_layer_kernel(D_HI, with_degw=False)
    lo = _make_layer_kernel(D_LO, with_degw=False)

    p0a, dw = lo_dw(src, dst, hW0a, ew0a, c0a, z_lo, z1,
                    jnp.reshape(ew2, (E_PAD,)))
    (p0b,) = hi(src, dst, hW0b, ew0b, c0b, z_hi)
    hW1a, hW1b, hWs1 = _tc_post(p0a, p0b, hWs0, node_attr, Wn1, Ws1, True)

    (p1a,) = lo(src, dst, hW1a, ew1a, c1a, z_lo)
    (p1b,) = hi(src, dst, hW1b, ew1b, c1b, z_hi)
    hW2, hWs2 = _tc_post(p1a, p1b, hWs1, node_attr, Wn2, Ws2, False)

    return _tc_final(hW2, hWs2, node_attr,
                     jnp.reshape(dw, (NC, N_PAD, 1)), c2)


# speedup vs baseline: 1.9823x; 1.0362x over previous
import functools
import math

import jax
import jax.numpy as jnp
import numpy as np
from jax import lax
from jax.experimental import pallas as pl
from jax.experimental.pallas import tpu as pltpu
from jax.experimental.pallas import tpu_sc as plsc

N = 10000
E = 160000
NB = 10
MAX_RADIUS = 2.0
D_IN = 16
D_EDGE = 4
D_SH = 9
D_EA = D_EDGE + D_SH
D_HID = 144
FC_HID = 100

NC = 2
NS = 16
NW = NC * NS
K = 128
E_PAD = 163840
NBLK0 = 50
NBLK1 = 30
N_PAD = 10240
ROWS_PER_SID = N_PAD // NS
ZCH = 64

_SQ3 = float(np.sqrt(3.0))
_SQ15 = float(np.sqrt(15.0))
_SQ5H = float(np.sqrt(5.0) / 2.0)
_SQ15H = float(np.sqrt(15.0) / 2.0)
_VALUES = np.linspace(0.0, MAX_RADIUS, NB + 2)[1:-1].astype(np.float32)
_STEP = float(_VALUES[1] - _VALUES[0])
_SQNB = float(np.sqrt(float(NB)))


def _mesh():
    return plsc.VectorSubcoreMesh(core_axis_name="c", subcore_axis_name="s")


_SC_PARAMS = pltpu.CompilerParams(needs_layout_passes=False,
                                  use_tc_tiling_on_sc=False)


def _make_edge_vec_kernel():
    @functools.partial(
        pl.kernel,
        mesh=_mesh(),
        out_type=jax.ShapeDtypeStruct((3, E_PAD), jnp.float32),
        compiler_params=_SC_PARAMS,
        scratch_types=[
            pltpu.VMEM((N_PAD * 3,), jnp.float32),
            pltpu.VMEM((K,), jnp.int32),
            pltpu.VMEM((K,), jnp.int32),
            pltpu.VMEM((3, K), jnp.float32),
        ],
    )
    def edge_vec_kernel(pos_hbm, src_hbm, dst_hbm, ev_hbm, pos_v, src_v, dst_v, out_v):
        cid = lax.axis_index("c")
        sid = lax.axis_index("s")
        wid = cid * NS + sid
        pltpu.sync_copy(pos_hbm, pos_v)

        def block_body(b, _):
            base = (wid * 40 + b) * K
            pltpu.sync_copy(src_hbm.at[pl.ds(base, K)], src_v)
            pltpu.sync_copy(dst_hbm.at[pl.ds(base, K)], dst_v)
            for k in range(K // 16):
                s3 = src_v[pl.ds(k * 16, 16)] * 3
                d3 = dst_v[pl.ds(k * 16, 16)] * 3
                for comp in range(3):
                    ps = plsc.load_gather(pos_v, [s3 + comp])
                    pd = plsc.load_gather(pos_v, [d3 + comp])
                    out_v[comp, pl.ds(k * 16, 16)] = ps - pd
            pltpu.sync_copy(out_v, ev_hbm.at[:, pl.ds(base, K)])
            return ()

        lax.fori_loop(0, 40, block_body, (), unroll=False)

    return edge_vec_kernel


D_LO = 80
D_HI = D_HID - D_LO


def _make_layer_kernel(d_c, with_degw):
    out_type = [jax.ShapeDtypeStruct((NC, N_PAD, d_c), jnp.float32)]
    if with_degw:
        out_type.append(jax.ShapeDtypeStruct((NC, N_PAD), jnp.float32))
    buf = [pltpu.VMEM((K,), jnp.int32),
           pltpu.VMEM((K,), jnp.int32),
           pltpu.VMEM((K, d_c), jnp.float32),
           pltpu.VMEM((K, d_c), jnp.float32),
           pltpu.VMEM((K, d_c), jnp.float32)]
    scratch = buf + buf + [
        pltpu.VMEM((ZCH, d_c), jnp.float32),
        pltpu.VMEM_SHARED((N_PAD, d_c), jnp.float32),
        pltpu.SemaphoreType.DMA,
        pltpu.SemaphoreType.DMA,
    ]
    if with_degw:
        scratch.append(pltpu.VMEM((K,), jnp.float32))
        scratch.append(pltpu.VMEM((K,), jnp.float32))
        scratch.append(pltpu.VMEM_SHARED((N_PAD,), jnp.float32))

    def body(*refs):
        if with_degw:
            (src_hbm, dst_hbm, hw_hbm, ew_hbm, c_hbm, z144_hbm, z1_hbm, ew2_hbm,
             agg_out, dw_out,
             src0, dst0, g0, ew0, c0, src1, dst1, g1, ew1, c1,
             z_v, agg_s, sem0, sem1, ew2_0, ew2_1, dw_s) = refs
        else:
            (src_hbm, dst_hbm, hw_hbm, ew_hbm, c_hbm, z144_hbm,
             agg_out,
             src0, dst0, g0, ew0, c0, src1, dst1, g1, ew1, c1,
             z_v, agg_s, sem0, sem1) = refs
            ew2_0 = ew2_1 = None
        cid = lax.axis_index("c")
        sid = lax.axis_index("s")
        nblk = jnp.where(cid == 0, NBLK0, NBLK1)
        blk_base = jnp.where(cid == 0, sid * NBLK0, NS * NBLK0 + sid * NBLK1)

        pltpu.sync_copy(z144_hbm, z_v)
        for j in range(ROWS_PER_SID // ZCH):
            pltpu.sync_copy(z_v, agg_s.at[pl.ds(sid * ROWS_PER_SID + j * ZCH, ZCH), :])
        if with_degw:
            @pl.when(sid == 0)
            def _():
                pltpu.sync_copy(z1_hbm, dw_s)
        plsc.subcore_barrier()

        def issue(b, src_v, dst_v, g_v, ew_v, c_v, ew2_v, sem):
            base = (blk_base + b) * K
            pltpu.sync_copy(src_hbm.at[pl.ds(base, K)], src_v)
            pltpu.sync_copy(dst_hbm.at[pl.ds(base, K)], dst_v)
            if with_degw:
                pltpu.sync_copy(ew2_hbm.at[pl.ds(base, K)], ew2_v)
            pltpu.async_copy(hw_hbm.at[src_v], g_v, sem)
            pltpu.async_copy(ew_hbm.at[pl.ds(base, K), :], ew_v, sem)
            pltpu.async_copy(c_hbm.at[pl.ds(base, K), :], c_v, sem)

        def consume(b, src_v, dst_v, g_v, ew_v, c_v, ew2_v, sem):
            base = (blk_base + b) * K
            pltpu.make_async_copy(hw_hbm.at[src_v], g_v, sem).wait()
            pltpu.make_async_copy(ew_hbm.at[pl.ds(base, K), :], ew_v, sem).wait()
            pltpu.make_async_copy(c_hbm.at[pl.ds(base, K), :], c_v, sem).wait()
            if with_degw:
                pltpu.sync_copy(ew2_v, dw_s.at[src_v], add=True)

            def edge_body(i, _):
                for j in range(d_c // 16):
                    sl = pl.ds(j * 16, 16)
                    g_v[i, sl] = g_v[i, sl] * ew_v[i, sl] + c_v[i, sl]
                return ()

            lax.fori_loop(0, K, edge_body, (), unroll=False)
            pltpu.sync_copy(g_v, agg_s.at[dst_v], add=True)

        buf0 = (src0, dst0, g0, ew0, c0, ew2_0, sem0)
        buf1 = (src1, dst1, g1, ew1, c1, ew2_1, sem1)
        issue(0, *buf0)

        def pair_body(t, _):
            b0 = 2 * t
            issue(b0 + 1, *buf1)
            consume(b0, *buf0)

            @pl.when(b0 + 2 < nblk)
            def _():
                issue(b0 + 2, *buf0)

            consume(b0 + 1, *buf1)
            return ()

        lax.fori_loop(0, nblk // 2, pair_body, (), unroll=False)
        plsc.subcore_barrier()
        rows = pl.ds(sid * ROWS_PER_SID, ROWS_PER_SID)
        pltpu.sync_copy(agg_s.at[rows, :], agg_out.at[cid, rows, :])
        if with_degw:
            pltpu.sync_copy(dw_s.at[rows], dw_out.at[cid, rows])

    return functools.partial(
        pl.kernel, mesh=_mesh(), out_type=out_type, scratch_types=scratch,
        compiler_params=_SC_PARAMS,
    )(body)


BE = 1024


def _bf(a):
    return a.astype(jnp.bfloat16)


def _geom_t(ev_ref, eat_ref, pid):
    v = ev_ref[...]
    x0, y0, z0 = v[0:1, :], v[1:2, :], v[2:3, :]
    length = jnp.sqrt(x0 * x0 + y0 * y0 + z0 * z0)
    inv = 1.0 / (length + 1e-9)
    x, y, z = x0 * inv, y0 * inv, z0 * inv
    sh = jnp.concatenate([
        jnp.ones_like(x), _SQ3 * x, _SQ3 * y, _SQ3 * z,
        _SQ15 * x * y, _SQ15 * y * z, _SQ5H * (3.0 * z * z - 1.0),
        _SQ15 * x * z, _SQ15H * (x * x - y * y),
    ], axis=0)
    ea = jnp.concatenate([eat_ref[...], sh], axis=0)
    vals = (lax.broadcasted_iota(jnp.int32, (NB, 1), 0).astype(jnp.float32)
            + 1.0) * _STEP
    diff = (length - vals) / _STEP
    emb = jnp.cos((np.pi / 2.0) * diff) * ((diff < 1.0) & (diff > -1.0)) * _SQNB
    ids = pid * BE + lax.broadcasted_iota(jnp.int32, (1, BE), 1)
    mask = (ids < E).astype(jnp.float32)
    return ea, emb, mask


def _coeffs(ea, emb, mask, Wf1, bf, Wf2, We):
    z = jnp.dot(_bf(emb), _bf(Wf1[...]), preferred_element_type=jnp.float32)
    s = jax.nn.silu(z + bf[...][None, :])
    ew = jnp.dot(_bf(s), _bf(Wf2[...]), preferred_element_type=jnp.float32)
    c = jnp.dot(_bf(ea), _bf(We[...]), preferred_element_type=jnp.float32) * ew
    return ew * mask, c * mask


def _tc_coeff_body(ev_ref, eat_ref,
                   Wf1_0, bf0, Wf2_0, We0, Wf1_1, bf1, Wf2_1, We1,
                   Wf1_2, bf2, Wf2_2, We2,
                   ew0a_ref, ew0b_ref, c0a_ref, c0b_ref,
                   ew1a_ref, ew1b_ref, c1a_ref, c1b_ref,
                   ew2_ref, c2_ref):
    pid = pl.program_id(0)
    eaT, embT, _ = _geom_t(ev_ref, eat_ref, pid)
    ea = jnp.transpose(eaT)
    emb = jnp.transpose(embT)
    ids = pid * BE + lax.broadcasted_iota(jnp.int32, (BE, 1), 0)
    mask = (ids < E).astype(jnp.float32)
    ew0, c0 = _coeffs(ea, emb, mask, Wf1_0, bf0, Wf2_0, We0)
    ew0a_ref[...], ew0b_ref[...] = ew0[:, :D_LO], ew0[:, D_LO:]
    c0a_ref[...], c0b_ref[...] = c0[:, :D_LO], c0[:, D_LO:]
    ew1, c1 = _coeffs(ea, emb, mask, Wf1_1, bf1, Wf2_1, We1)
    ew1a_ref[...], ew1b_ref[...] = ew1[:, :D_LO], ew1[:, D_LO:]
    c1a_ref[...], c1b_ref[...] = c1[:, :D_LO], c1[:, D_LO:]
    ew2, c2 = _coeffs(ea, emb, mask, Wf1_2, bf2, Wf2_2, We2)
    t = jnp.transpose(jnp.concatenate([ew2, c2], axis=1))
    ew2_ref[...] = t[0:1, :][None]
    c2_ref[...] = t[1:2, :][None]


def _tc_coeffs(ev, eat, Wf1_0, bf0, Wf2_0, We0, Wf1_1, bf1, Wf2_1, We1,
               Wf1_2, bf2, Wf2_2, We2):
    full = lambda shape: pl.BlockSpec(shape, lambda i: tuple(0 for _ in shape))
    row = lambda d: pl.BlockSpec((BE, d), lambda i: (i, 0))
    col = lambda d: pl.BlockSpec((d, BE), lambda i: (0, i))
    return pl.pallas_call(
        _tc_coeff_body,
        grid=(E_PAD // BE,),
        in_specs=[col(3), col(D_EDGE),
                  full((NB, FC_HID)), full((FC_HID,)), full((FC_HID, D_HID)),
                  full((D_EA, D_HID)),
                  full((NB, FC_HID)), full((FC_HID,)), full((FC_HID, D_HID)),
                  full((D_EA, D_HID)),
                  full((NB, FC_HID)), full((FC_HID,)), full((FC_HID, 1)),
                  full((D_EA, 1))],
        out_specs=[row(D_LO), row(D_HI), row(D_LO), row(D_HI),
                   row(D_LO), row(D_HI), row(D_LO), row(D_HI),
                   pl.BlockSpec((1, 1, BE), lambda i: (i, 0, 0)),
                   pl.BlockSpec((1, 1, BE), lambda i: (i, 0, 0))],
        out_shape=[jax.ShapeDtypeStruct((E_PAD, D_LO), jnp.float32),
                   jax.ShapeDtypeStruct((E_PAD, D_HI), jnp.float32)] * 4
        + [jax.ShapeDtypeStruct((E_PAD // BE, 1, BE), jnp.float32)] * 2,
    )(ev, eat, Wf1_0, bf0, Wf2_0, We0, Wf1_1, bf1, Wf2_1, We1,
      Wf1_2, bf2, Wf2_2, We2)


NROW = 1000


def _split_outs(hW, hWs, o1a_ref, o1b_ref, o2_ref):
    o1a_ref[...], o1b_ref[...] = hW[:, :D_LO], hW[:, D_LO:]
    o2_ref[...] = hWs


def _node_out_specs(dout, split):
    if split:
        return ([pl.BlockSpec((NROW, D_LO), lambda i: (i, 0)),
                 pl.BlockSpec((NROW, D_HI), lambda i: (i, 0)),
                 pl.BlockSpec((NROW, dout), lambda i: (i, 0))],
                [jax.ShapeDtypeStruct((N, D_LO), jnp.float32),
                 jax.ShapeDtypeStruct((N, D_HI), jnp.float32),
                 jax.ShapeDtypeStruct((N, dout), jnp.float32)])
    return ([pl.BlockSpec((NROW, dout), lambda i: (i, 0))] * 2,
            [jax.ShapeDtypeStruct((N, dout), jnp.float32)] * 2)


def _tc_pre(h, Wn, Ws):
    din, dout = Wn.shape
    full = lambda shape: pl.BlockSpec(shape, lambda i: tuple(0 for _ in shape))
    out_specs, out_shape = _node_out_specs(dout, True)

    def body(h_ref, Wn_ref, Ws_ref, o1a_ref, o1b_ref, o2_ref):
        h = h_ref[...]
        hW = jnp.dot(h, Wn_ref[...], preferred_element_type=jnp.float32)
        hWs = jnp.dot(h, Ws_ref[...], preferred_element_type=jnp.float32)
        _split_outs(hW, hWs, o1a_ref, o1b_ref, o2_ref)

    return pl.pallas_call(
        body,
        grid=(N // NROW,),
        in_specs=[pl.BlockSpec((NROW, din), lambda i: (i, 0)),
                  full((din, dout)), full((din, dout))],
        out_specs=out_specs, out_shape=out_shape,
    )(h, Wn, Ws)


def _tc_post(pa, pb, hWs, attr, Wn, Ws, split):
    din, dout = Wn.shape
    full = lambda shape: pl.BlockSpec(shape, lambda i: tuple(0 for _ in shape))
    out_specs, out_shape = _node_out_specs(dout, split)

    def body(pa_ref, pb_ref, hWs_ref, attr_ref, Wn_ref, Ws_ref, *outs):
        agg = jnp.concatenate([pa_ref[0] + pa_ref[1],
                               pb_ref[0] + pb_ref[1]], axis=1) * 0.25
        h = jax.nn.silu(hWs_ref[...] * attr_ref[...] + agg)
        hW = jnp.dot(h, Wn_ref[...], preferred_element_type=jnp.float32)
        hWs2 = jnp.dot(h, Ws_ref[...], preferred_element_type=jnp.float32)
        if split:
            _split_outs(hW, hWs2, *outs)
        else:
            outs[0][...] = hW
            outs[1][...] = hWs2

    return pl.pallas_call(
        body,
        grid=(N // NROW,),
        in_specs=[pl.BlockSpec((NC, NROW, D_LO), lambda i: (0, i, 0)),
                  pl.BlockSpec((NC, NROW, D_HI), lambda i: (0, i, 0)),
                  pl.BlockSpec((NROW, din), lambda i: (i, 0)),
                  pl.BlockSpec((NROW, 1), lambda i: (i, 0)),
                  full((din, dout)), full((din, dout))],
        out_specs=out_specs, out_shape=out_shape,
    )(pa, pb, hWs, attr, Wn, Ws)


def _tc_final_body(hW2_ref, hWs2_ref, attr_ref, dw_ref, c2_ref, out_ref):
    term1 = jnp.sum(hWs2_ref[...] * attr_ref[...])
    degw = dw_ref[0, :N, :] + dw_ref[1, :N, :]
    term2 = jnp.sum(hW2_ref[...] * degw)
    term3 = jnp.sum(c2_ref[...])
    total = (term1 + (term2 + term3) * 0.25) / 100.0
    out_ref[...] = jnp.reshape(total, (1, 1))


def _tc_final(hW2, hWs2, attr, dw, c2):
    full = lambda shape: pl.BlockSpec(shape, lambda: tuple(0 for _ in shape))
    return pl.pallas_call(
        _tc_final_body,
        in_specs=[full((N, 1)), full((N, 1)), full((N, 1)),
                  full((NC, N_PAD, 1)), full((E_PAD // BE, 1, BE))],
        out_specs=full((1, 1)),
        out_shape=jax.ShapeDtypeStruct((1, 1), jnp.float32),
    )(hW2, hWs2, attr, dw, c2)


def kernel(pos, node_input, node_attr, edge_attr, edge_index, batch,
           Wn0, We0, Wf1_0, bf0, Wf2_0, Ws0,
           Wn1, We1, Wf1_1, bf1, Wf2_1, Ws1,
           Wn2, We2, Wf1_2, bf2, Wf2_2, Ws2):
    pad = E_PAD - E
    src = jnp.concatenate([edge_index[0], jnp.zeros((pad,), jnp.int32)])
    dst = jnp.concatenate([edge_index[1], jnp.full((pad,), N, jnp.int32)])
    eat = jnp.concatenate([jnp.transpose(edge_attr),
                           jnp.zeros((D_EDGE, pad), jnp.float32)], axis=1)
    pos_pad = jnp.concatenate([pos, jnp.zeros((N_PAD - N, 3), jnp.float32)])
    z_lo = jnp.zeros((ZCH, D_LO), jnp.float32)
    z_hi = jnp.zeros((ZCH, D_HI), jnp.float32)
    z1 = jnp.zeros((N_PAD,), jnp.float32)

    ev = _make_edge_vec_kernel()(jnp.reshape(pos_pad, (N_PAD * 3,)), src, dst)
    (ew0a, ew0b, c0a, c0b, ew1a, ew1b, c1a, c1b, ew2, c2) = _tc_coeffs(
        ev, eat, Wf1_0, bf0, Wf2_0, We0, Wf1_1, bf1, Wf2_1, We1,
        Wf1_2, bf2, Wf2_2, We2)
    hW0a, hW0b, hWs0 = _tc_pre(node_input, Wn0, Ws0)

    lo_dw = _make_layer_kernel(D_LO, with_degw=True)
    hi = _make_layer_kernel(D_HI, with_degw=False)
    lo = _make_layer_kernel(D_LO, with_degw=False)

    p0a, dw = lo_dw(src, dst, hW0a, ew0a, c0a, z_lo, z1,
                    jnp.reshape(ew2, (E_PAD,)))
    (p0b,) = hi(src, dst, hW0b, ew0b, c0b, z_hi)
    hW1a, hW1b, hWs1 = _tc_post(p0a, p0b, hWs0, node_attr, Wn1, Ws1, True)

    (p1a,) = lo(src, dst, hW1a, ew1a, c1a, z_lo)
    (p1b,) = hi(src, dst, hW1b, ew1b, c1b, z_hi)
    hW2, hWs2 = _tc_post(p1a, p1b, hWs1, node_attr, Wn2, Ws2, False)

    return _tc_final(hW2, hWs2, node_attr,
                     jnp.reshape(dw, (NC, N_PAD, 1)), c2)


# speedup vs baseline: 1.9910x; 1.0044x over previous
import functools
import math

import jax
import jax.numpy as jnp
import numpy as np
from jax import lax
from jax.experimental import pallas as pl
from jax.experimental.pallas import tpu as pltpu
from jax.experimental.pallas import tpu_sc as plsc

N = 10000
E = 160000
NB = 10
MAX_RADIUS = 2.0
D_IN = 16
D_EDGE = 4
D_SH = 9
D_EA = D_EDGE + D_SH
D_HID = 144
FC_HID = 100

NC = 2
NS = 16
NW = NC * NS
K = 128
E_PAD = 163840
NBLK0 = 56
NBLK1 = 24
N_PAD = 10240
ROWS_PER_SID = N_PAD // NS
ZCH = 64

_SQ3 = float(np.sqrt(3.0))
_SQ15 = float(np.sqrt(15.0))
_SQ5H = float(np.sqrt(5.0) / 2.0)
_SQ15H = float(np.sqrt(15.0) / 2.0)
_VALUES = np.linspace(0.0, MAX_RADIUS, NB + 2)[1:-1].astype(np.float32)
_STEP = float(_VALUES[1] - _VALUES[0])
_SQNB = float(np.sqrt(float(NB)))


def _mesh():
    return plsc.VectorSubcoreMesh(core_axis_name="c", subcore_axis_name="s")


_SC_PARAMS = pltpu.CompilerParams(needs_layout_passes=False,
                                  use_tc_tiling_on_sc=False)


def _make_edge_vec_kernel():
    @functools.partial(
        pl.kernel,
        mesh=_mesh(),
        out_type=jax.ShapeDtypeStruct((3, E_PAD), jnp.float32),
        compiler_params=_SC_PARAMS,
        scratch_types=[
            pltpu.VMEM((N_PAD * 3,), jnp.float32),
            pltpu.VMEM((K,), jnp.int32),
            pltpu.VMEM((K,), jnp.int32),
            pltpu.VMEM((3, K), jnp.float32),
        ],
    )
    def edge_vec_kernel(pos_hbm, src_hbm, dst_hbm, ev_hbm, pos_v, src_v, dst_v, out_v):
        cid = lax.axis_index("c")
        sid = lax.axis_index("s")
        wid = cid * NS + sid
        pltpu.sync_copy(pos_hbm, pos_v)

        def block_body(b, _):
            base = (wid * 40 + b) * K
            pltpu.sync_copy(src_hbm.at[pl.ds(base, K)], src_v)
            pltpu.sync_copy(dst_hbm.at[pl.ds(base, K)], dst_v)
            for k in range(K // 16):
                s3 = src_v[pl.ds(k * 16, 16)] * 3
                d3 = dst_v[pl.ds(k * 16, 16)] * 3
                for comp in range(3):
                    ps = plsc.load_gather(pos_v, [s3 + comp])
                    pd = plsc.load_gather(pos_v, [d3 + comp])
                    out_v[comp, pl.ds(k * 16, 16)] = ps - pd
            pltpu.sync_copy(out_v, ev_hbm.at[:, pl.ds(base, K)])
            return ()

        lax.fori_loop(0, 40, block_body, (), unroll=False)

    return edge_vec_kernel


D_LO = 80
D_HI = D_HID - D_LO


def _make_layer_kernel(d_c, with_degw):
    out_type = [jax.ShapeDtypeStruct((NC, N_PAD, d_c), jnp.float32)]
    if with_degw:
        out_type.append(jax.ShapeDtypeStruct((NC, N_PAD), jnp.float32))
    buf = [pltpu.VMEM((K,), jnp.int32),
           pltpu.VMEM((K,), jnp.int32),
           pltpu.VMEM((K, d_c), jnp.float32),
           pltpu.VMEM((K, d_c), jnp.float32),
           pltpu.VMEM((K, d_c), jnp.float32)]
    scratch = buf + buf + [
        pltpu.VMEM((ZCH, d_c), jnp.float32),
        pltpu.VMEM_SHARED((N_PAD, d_c), jnp.float32),
        pltpu.SemaphoreType.DMA,
        pltpu.SemaphoreType.DMA,
    ]
    if with_degw:
        scratch.append(pltpu.VMEM((K,), jnp.float32))
        scratch.append(pltpu.VMEM((K,), jnp.float32))
        scratch.append(pltpu.VMEM_SHARED((N_PAD,), jnp.float32))

    def body(*refs):
        if with_degw:
            (src_hbm, dst_hbm, hw_hbm, ew_hbm, c_hbm, z144_hbm, z1_hbm, ew2_hbm,
             agg_out, dw_out,
             src0, dst0, g0, ew0, c0, src1, dst1, g1, ew1, c1,
             z_v, agg_s, sem0, sem1, ew2_0, ew2_1, dw_s) = refs
        else:
            (src_hbm, dst_hbm, hw_hbm, ew_hbm, c_hbm, z144_hbm,
             agg_out,
             src0, dst0, g0, ew0, c0, src1, dst1, g1, ew1, c1,
             z_v, agg_s, sem0, sem1) = refs
            ew2_0 = ew2_1 = None
        cid = lax.axis_index("c")
        sid = lax.axis_index("s")
        nblk = jnp.where(cid == 0, NBLK0, NBLK1)
        blk_base = jnp.where(cid == 0, sid * NBLK0, NS * NBLK0 + sid * NBLK1)

        pltpu.sync_copy(z144_hbm, z_v)
        for j in range(ROWS_PER_SID // ZCH):
            pltpu.sync_copy(z_v, agg_s.at[pl.ds(sid * ROWS_PER_SID + j * ZCH, ZCH), :])
        if with_degw:
            @pl.when(sid == 0)
            def _():
                pltpu.sync_copy(z1_hbm, dw_s)
        plsc.subcore_barrier()

        def issue(b, src_v, dst_v, g_v, ew_v, c_v, ew2_v, sem):
            base = (blk_base + b) * K
            pltpu.sync_copy(src_hbm.at[pl.ds(base, K)], src_v)
            pltpu.sync_copy(dst_hbm.at[pl.ds(base, K)], dst_v)
            if with_degw:
                pltpu.sync_copy(ew2_hbm.at[pl.ds(base, K)], ew2_v)
            pltpu.async_copy(hw_hbm.at[src_v], g_v, sem)
            pltpu.async_copy(ew_hbm.at[pl.ds(base, K), :], ew_v, sem)
            pltpu.async_copy(c_hbm.at[pl.ds(base, K), :], c_v, sem)

        def consume(b, src_v, dst_v, g_v, ew_v, c_v, ew2_v, sem):
            base = (blk_base + b) * K
            pltpu.make_async_copy(hw_hbm.at[src_v], g_v, sem).wait()
            pltpu.make_async_copy(ew_hbm.at[pl.ds(base, K), :], ew_v, sem).wait()
            pltpu.make_async_copy(c_hbm.at[pl.ds(base, K), :], c_v, sem).wait()
            if with_degw:
                pltpu.sync_copy(ew2_v, dw_s.at[src_v], add=True)

            def edge_body(i, _):
                for j in range(d_c // 16):
                    sl = pl.ds(j * 16, 16)
                    g_v[i, sl] = g_v[i, sl] * ew_v[i, sl] + c_v[i, sl]
                return ()

            lax.fori_loop(0, K, edge_body, (), unroll=False)
            pltpu.sync_copy(g_v, agg_s.at[dst_v], add=True)

        buf0 = (src0, dst0, g0, ew0, c0, ew2_0, sem0)
        buf1 = (src1, dst1, g1, ew1, c1, ew2_1, sem1)
        issue(0, *buf0)

        def pair_body(t, _):
            b0 = 2 * t
            issue(b0 + 1, *buf1)
            consume(b0, *buf0)

            @pl.when(b0 + 2 < nblk)
            def _():
                issue(b0 + 2, *buf0)

            consume(b0 + 1, *buf1)
            return ()

        lax.fori_loop(0, nblk // 2, pair_body, (), unroll=False)
        plsc.subcore_barrier()
        rows = pl.ds(sid * ROWS_PER_SID, ROWS_PER_SID)
        pltpu.sync_copy(agg_s.at[rows, :], agg_out.at[cid, rows, :])
        if with_degw:
            pltpu.sync_copy(dw_s.at[rows], dw_out.at[cid, rows])

    return functools.partial(
        pl.kernel, mesh=_mesh(), out_type=out_type, scratch_types=scratch,
        compiler_params=_SC_PARAMS,
    )(body)


BE = 1024


def _bf(a):
    return a.astype(jnp.bfloat16)


def _geom_t(ev_ref, eat_ref, pid):
    v = ev_ref[...]
    x0, y0, z0 = v[0:1, :], v[1:2, :], v[2:3, :]
    length = jnp.sqrt(x0 * x0 + y0 * y0 + z0 * z0)
    inv = 1.0 / (length + 1e-9)
    x, y, z = x0 * inv, y0 * inv, z0 * inv
    sh = jnp.concatenate([
        jnp.ones_like(x), _SQ3 * x, _SQ3 * y, _SQ3 * z,
        _SQ15 * x * y, _SQ15 * y * z, _SQ5H * (3.0 * z * z - 1.0),
        _SQ15 * x * z, _SQ15H * (x * x - y * y),
    ], axis=0)
    ea = jnp.concatenate([eat_ref[...], sh], axis=0)
    vals = (lax.broadcasted_iota(jnp.int32, (NB, 1), 0).astype(jnp.float32)
            + 1.0) * _STEP
    diff = (length - vals) / _STEP
    emb = jnp.cos((np.pi / 2.0) * diff) * ((diff < 1.0) & (diff > -1.0)) * _SQNB
    ids = pid * BE + lax.broadcasted_iota(jnp.int32, (1, BE), 1)
    mask = (ids < E).astype(jnp.float32)
    return ea, emb, mask


def _coeffs(ea, emb, mask, Wf1, bf, Wf2, We):
    z = jnp.dot(_bf(emb), _bf(Wf1[...]), preferred_element_type=jnp.float32)
    s = jax.nn.silu(z + bf[...][None, :])
    ew = jnp.dot(_bf(s), _bf(Wf2[...]), preferred_element_type=jnp.float32)
    c = jnp.dot(_bf(ea), _bf(We[...]), preferred_element_type=jnp.float32) * ew
    return ew * mask, c * mask


def _tc_coeff_body(ev_ref, eat_ref,
                   Wf1_0, bf0, Wf2_0, We0, Wf1_1, bf1, Wf2_1, We1,
                   Wf1_2, bf2, Wf2_2, We2,
                   ew0a_ref, ew0b_ref, c0a_ref, c0b_ref,
                   ew1a_ref, ew1b_ref, c1a_ref, c1b_ref,
                   ew2_ref, c2_ref):
    pid = pl.program_id(0)
    eaT, embT, _ = _geom_t(ev_ref, eat_ref, pid)
    ea = jnp.transpose(eaT)
    emb = jnp.transpose(embT)
    ids = pid * BE + lax.broadcasted_iota(jnp.int32, (BE, 1), 0)
    mask = (ids < E).astype(jnp.float32)
    ew0, c0 = _coeffs(ea, emb, mask, Wf1_0, bf0, Wf2_0, We0)
    ew0a_ref[...], ew0b_ref[...] = ew0[:, :D_LO], ew0[:, D_LO:]
    c0a_ref[...], c0b_ref[...] = c0[:, :D_LO], c0[:, D_LO:]
    ew1, c1 = _coeffs(ea, emb, mask, Wf1_1, bf1, Wf2_1, We1)
    ew1a_ref[...], ew1b_ref[...] = ew1[:, :D_LO], ew1[:, D_LO:]
    c1a_ref[...], c1b_ref[...] = c1[:, :D_LO], c1[:, D_LO:]
    ew2, c2 = _coeffs(ea, emb, mask, Wf1_2, bf2, Wf2_2, We2)
    t = jnp.transpose(jnp.concatenate([ew2, c2], axis=1))
    ew2_ref[...] = t[0:1, :][None]
    c2_ref[...] = t[1:2, :][None]


def _tc_coeffs(ev, eat, Wf1_0, bf0, Wf2_0, We0, Wf1_1, bf1, Wf2_1, We1,
               Wf1_2, bf2, Wf2_2, We2):
    full = lambda shape: pl.BlockSpec(shape, lambda i: tuple(0 for _ in shape))
    row = lambda d: pl.BlockSpec((BE, d), lambda i: (i, 0))
    col = lambda d: pl.BlockSpec((d, BE), lambda i: (0, i))
    return pl.pallas_call(
        _tc_coeff_body,
        grid=(E_PAD // BE,),
        in_specs=[col(3), col(D_EDGE),
                  full((NB, FC_HID)), full((FC_HID,)), full((FC_HID, D_HID)),
                  full((D_EA, D_HID)),
                  full((NB, FC_HID)), full((FC_HID,)), full((FC_HID, D_HID)),
                  full((D_EA, D_HID)),
                  full((NB, FC_HID)), full((FC_HID,)), full((FC_HID, 1)),
                  full((D_EA, 1))],
        out_specs=[row(D_LO), row(D_HI), row(D_LO), row(D_HI),
                   row(D_LO), row(D_HI), row(D_LO), row(D_HI),
                   pl.BlockSpec((1, 1, BE), lambda i: (i, 0, 0)),
                   pl.BlockSpec((1, 1, BE), lambda i: (i, 0, 0))],
        out_shape=[jax.ShapeDtypeStruct((E_PAD, D_LO), jnp.float32),
                   jax.ShapeDtypeStruct((E_PAD, D_HI), jnp.float32)] * 4
        + [jax.ShapeDtypeStruct((E_PAD // BE, 1, BE), jnp.float32)] * 2,
    )(ev, eat, Wf1_0, bf0, Wf2_0, We0, Wf1_1, bf1, Wf2_1, We1,
      Wf1_2, bf2, Wf2_2, We2)


NROW = 1000


def _split_outs(hW, hWs, o1a_ref, o1b_ref, o2_ref):
    o1a_ref[...], o1b_ref[...] = hW[:, :D_LO], hW[:, D_LO:]
    o2_ref[...] = hWs


def _node_out_specs(dout, split):
    if split:
        return ([pl.BlockSpec((NROW, D_LO), lambda i: (i, 0)),
                 pl.BlockSpec((NROW, D_HI), lambda i: (i, 0)),
                 pl.BlockSpec((NROW, dout), lambda i: (i, 0))],
                [jax.ShapeDtypeStruct((N, D_LO), jnp.float32),
                 jax.ShapeDtypeStruct((N, D_HI), jnp.float32),
                 jax.ShapeDtypeStruct((N, dout), jnp.float32)])
    return ([pl.BlockSpec((NROW, dout), lambda i: (i, 0))] * 2,
            [jax.ShapeDtypeStruct((N, dout), jnp.float32)] * 2)


def _tc_pre(h, Wn, Ws):
    din, dout = Wn.shape
    full = lambda shape: pl.BlockSpec(shape, lambda i: tuple(0 for _ in shape))
    out_specs, out_shape = _node_out_specs(dout, True)

    def body(h_ref, Wn_ref, Ws_ref, o1a_ref, o1b_ref, o2_ref):
        h = h_ref[...]
        hW = jnp.dot(h, Wn_ref[...], preferred_element_type=jnp.float32)
        hWs = jnp.dot(h, Ws_ref[...], preferred_element_type=jnp.float32)
        _split_outs(hW, hWs, o1a_ref, o1b_ref, o2_ref)

    return pl.pallas_call(
        body,
        grid=(N // NROW,),
        in_specs=[pl.BlockSpec((NROW, din), lambda i: (i, 0)),
                  full((din, dout)), full((din, dout))],
        out_specs=out_specs, out_shape=out_shape,
    )(h, Wn, Ws)


def _tc_post(pa, pb, hWs, attr, Wn, Ws, split):
    din, dout = Wn.shape
    full = lambda shape: pl.BlockSpec(shape, lambda i: tuple(0 for _ in shape))
    out_specs, out_shape = _node_out_specs(dout, split)

    def body(pa_ref, pb_ref, hWs_ref, attr_ref, Wn_ref, Ws_ref, *outs):
        agg = jnp.concatenate([pa_ref[0] + pa_ref[1],
                               pb_ref[0] + pb_ref[1]], axis=1) * 0.25
        h = jax.nn.silu(hWs_ref[...] * attr_ref[...] + agg)
        hW = jnp.dot(h, Wn_ref[...], preferred_element_type=jnp.float32)
        hWs2 = jnp.dot(h, Ws_ref[...], preferred_element_type=jnp.float32)
        if split:
            _split_outs(hW, hWs2, *outs)
        else:
            outs[0][...] = hW
            outs[1][...] = hWs2

    return pl.pallas_call(
        body,
        grid=(N // NROW,),
        in_specs=[pl.BlockSpec((NC, NROW, D_LO), lambda i: (0, i, 0)),
                  pl.BlockSpec((NC, NROW, D_HI), lambda i: (0, i, 0)),
                  pl.BlockSpec((NROW, din), lambda i: (i, 0)),
                  pl.BlockSpec((NROW, 1), lambda i: (i, 0)),
                  full((din, dout)), full((din, dout))],
        out_specs=out_specs, out_shape=out_shape,
    )(pa, pb, hWs, attr, Wn, Ws)


def _tc_final_body(hW2_ref, hWs2_ref, attr_ref, dw_ref, c2_ref, out_ref):
    term1 = jnp.sum(hWs2_ref[...] * attr_ref[...])
    degw = dw_ref[0, :N, :] + dw_ref[1, :N, :]
    term2 = jnp.sum(hW2_ref[...] * degw)
    term3 = jnp.sum(c2_ref[...])
    total = (term1 + (term2 + term3) * 0.25) / 100.0
    out_ref[...] = jnp.reshape(total, (1, 1))


def _tc_final(hW2, hWs2, attr, dw, c2):
    full = lambda shape: pl.BlockSpec(shape, lambda: tuple(0 for _ in shape))
    return pl.pallas_call(
        _tc_final_body,
        in_specs=[full((N, 1)), full((N, 1)), full((N, 1)),
                  full((NC, N_PAD, 1)), full((E_PAD // BE, 1, BE))],
        out_specs=full((1, 1)),
        out_shape=jax.ShapeDtypeStruct((1, 1), jnp.float32),
    )(hW2, hWs2, attr, dw, c2)


def kernel(pos, node_input, node_attr, edge_attr, edge_index, batch,
           Wn0, We0, Wf1_0, bf0, Wf2_0, Ws0,
           Wn1, We1, Wf1_1, bf1, Wf2_1, Ws1,
           Wn2, We2, Wf1_2, bf2, Wf2_2, Ws2):
    pad = E_PAD - E
    src = jnp.concatenate([edge_index[0], jnp.zeros((pad,), jnp.int32)])
    dst = jnp.concatenate([edge_index[1], jnp.full((pad,), N, jnp.int32)])
    eat = jnp.concatenate([jnp.transpose(edge_attr),
                           jnp.zeros((D_EDGE, pad), jnp.float32)], axis=1)
    pos_pad = jnp.concatenate([pos, jnp.zeros((N_PAD - N, 3), jnp.float32)])
    z_lo = jnp.zeros((ZCH, D_LO), jnp.float32)
    z_hi = jnp.zeros((ZCH, D_HI), jnp.float32)
    z1 = jnp.zeros((N_PAD,), jnp.float32)

    ev = _make_edge_vec_kernel()(jnp.reshape(pos_pad, (N_PAD * 3,)), src, dst)
    (ew0a, ew0b, c0a, c0b, ew1a, ew1b, c1a, c1b, ew2, c2) = _tc_coeffs(
        ev, eat, Wf1_0, bf0, Wf2_0, We0, Wf1_1, bf1, Wf2_1, We1,
        Wf1_2, bf2, Wf2_2, We2)
    hW0a, hW0b, hWs0 = _tc_pre(node_input, Wn0, Ws0)

    lo_dw = _make_layer_kernel(D_LO, with_degw=True)
    hi = _make_layer_kernel(D_HI, with_degw=False)
    lo = _make_layer_kernel(D_LO, with_degw=False)

    p0a, dw = lo_dw(src, dst, hW0a, ew0a, c0a, z_lo, z1,
                    jnp.reshape(ew2, (E_PAD,)))
    (p0b,) = hi(src, dst, hW0b, ew0b, c0b, z_hi)
    hW1a, hW1b, hWs1 = _tc_post(p0a, p0b, hWs0, node_attr, Wn1, Ws1, True)

    (p1a,) = lo(src, dst, hW1a, ew1a, c1a, z_lo)
    (p1b,) = hi(src, dst, hW1b, ew1b, c1b, z_hi)
    hW2, hWs2 = _tc_post(p1a, p1b, hWs1, node_attr, Wn2, Ws2, False)

    return _tc_final(hW2, hWs2, node_attr,
                     jnp.reshape(dw, (NC, N_PAD, 1)), c2)
